# Initial kernel scaffold; baseline (speedup 1.0000x reference)
import jax
import jax.numpy as jnp
from jax.experimental import pallas as pl


def kernel(idx, targets, table):
    raise NotImplementedError("write your pallas kernel here")


# speedup vs baseline: 2.0751x; 2.0751x over previous
import functools

import jax
import jax.numpy as jnp
from jax import lax
from jax.experimental import pallas as pl
from jax.experimental.pallas import tpu as pltpu
from jax.experimental.pallas import tpu_sc as plsc

VOCAB = 4096
N_TOK = 8 * 2048
NC, NS, L = 2, 16, 16
NW = NC * NS
TPW = N_TOK // NW
CH = 4
NBUF = 4
NCHUNK = TPW // CH
NOUTER = NCHUNK // NBUF
JL = TPW // L


_LSE_RB = 256


def _lse_body(tbl_ref, lse_ref):
    x = tbl_ref[...]
    m = jnp.max(x, axis=-1, keepdims=True)
    s = jnp.sum(jnp.exp(x - m), axis=-1, keepdims=True)
    lse_ref[...] = m + jnp.log(s)


_lse_call = pl.pallas_call(
    _lse_body,
    grid=(VOCAB // _LSE_RB,),
    in_specs=[pl.BlockSpec((_LSE_RB, VOCAB), lambda i: (i, 0))],
    out_specs=pl.BlockSpec((_LSE_RB, 1), lambda i: (i, 0)),
    out_shape=jax.ShapeDtypeStruct((VOCAB, 1), jnp.float32),
)


_sc_mesh = plsc.VectorSubcoreMesh(
    core_axis_name="c", subcore_axis_name="s", num_cores=NC, num_subcores=NS
)


@functools.partial(
    pl.kernel,
    out_type=(
        jax.ShapeDtypeStruct((N_TOK, VOCAB), jnp.float32),
        jax.ShapeDtypeStruct((NW, L), jnp.float32),
    ),
    mesh=_sc_mesh,
    compiler_params=pltpu.CompilerParams(needs_layout_passes=False),
    scratch_types=[
        pltpu.VMEM((NBUF, CH, VOCAB), jnp.float32),
        pltpu.VMEM((NCHUNK, CH), jnp.int32),
        pltpu.VMEM((JL, L), jnp.int32),
        pltpu.VMEM((JL, L), jnp.int32),
        pltpu.VMEM((TPW,), jnp.int32),
        pltpu.VMEM((TPW,), jnp.float32),
        pltpu.VMEM((VOCAB,), jnp.float32),
        pltpu.VMEM((L,), jnp.float32),
        pltpu.SemaphoreType.DMA,
        pltpu.SemaphoreType.DMA,
        pltpu.SemaphoreType.DMA,
        pltpu.SemaphoreType.DMA,
        pltpu.SemaphoreType.DMA,
        pltpu.SemaphoreType.DMA,
        pltpu.SemaphoreType.DMA,
        pltpu.SemaphoreType.DMA,
        pltpu.SemaphoreType.DMA,
    ],
)
def _sc_call(table, tflat, idxg, idxl, tgt, lse, logits_out, part_out,
             rows_v, idxg_v, idxl_v, tgt_v, fi_v, picked_v, lse_v, acc_v,
             g0, g1, g2, g3, w0, w1, w2, w3, lsem):
    gsems = (g0, g1, g2, g3)
    wsems = (w0, w1, w2, w3)
    wid = lax.axis_index("s") * NC + lax.axis_index("c")
    base = wid * TPW

    def g_start(c, slot):
        pltpu.async_copy(table.at[idxg_v.at[c]], rows_v.at[slot], gsems[slot])

    def g_wait(c, slot):
        pltpu.make_async_copy(
            table.at[idxg_v.at[c]], rows_v.at[slot], gsems[slot]
        ).wait()

    def w_start(c, slot):
        pltpu.async_copy(
            rows_v.at[slot],
            logits_out.at[pl.ds(base + c * CH, CH)],
            wsems[slot],
        )

    def w_wait(c, slot):
        pltpu.make_async_copy(
            rows_v.at[slot],
            logits_out.at[pl.ds(base + c * CH, CH)],
            wsems[slot],
        ).wait()

    pltpu.sync_copy(idxg.at[wid], idxg_v)
    g_start(0, 0)
    g_start(1, 1)

    pltpu.sync_copy(idxl.at[wid], idxl_v)
    pltpu.sync_copy(tgt.at[wid], tgt_v)
    pltpu.sync_copy(lse, lse_v)
    for j in range(JL):
        iv = idxl_v[j]
        tv = tgt_v[j]
        fi_v[pl.ds(j * L, L)] = iv * VOCAB + tv
    pltpu.async_copy(tflat.at[fi_v], picked_v, lsem).wait()
    acc = jnp.zeros((L,), jnp.float32)
    for j in range(JL):
        lg = plsc.load_gather(lse_v, [idxl_v[j]])
        pv = picked_v[pl.ds(j * L, L)]
        acc = acc + (lg - pv)
    acc_v[...] = acc
    pltpu.sync_copy(acc_v, part_out.at[wid])

    def outer(t, carry):
        for b in range(NBUF):
            g = t * NBUF + b
            g_wait(g, b)
            w_start(g, b)
            nb = (b + 2) % NBUF
            if b < 2:
                @pl.when(t >= 1)
                def _():
                    w_wait(g - 2, nb)
                g_start(g + 2, nb)
            else:
                @pl.when(t < NOUTER - 1)
                def _():
                    w_wait(g - 2, nb)
                    g_start(g + 2, nb)
        return carry

    lax.fori_loop(0, NOUTER, outer, 0)
    for b in range(NBUF):
        w_wait(NCHUNK - NBUF + b, b)


def kernel(idx, targets, table):
    idx_flat = idx.reshape(-1).astype(jnp.int32)
    tgt_flat = targets.reshape(-1).astype(jnp.int32)
    lse = _lse_call(table).reshape(VOCAB)
    idxg = idx_flat.reshape(NW, NCHUNK, CH)
    idxl = idx_flat.reshape(NW, JL, L)
    tgt3 = tgt_flat.reshape(NW, JL, L)
    tflat = table.reshape(-1)
    logits, partials = _sc_call(table, tflat, idxg, idxl, tgt3, lse)
    loss = jnp.sum(partials) * (1.0 / N_TOK)
    return (logits, loss)


# speedup vs baseline: 2.3647x; 1.1396x over previous
import functools

import jax
import jax.numpy as jnp
from jax import lax
from jax.experimental import pallas as pl
from jax.experimental.pallas import tpu as pltpu
from jax.experimental.pallas import tpu_sc as plsc

VOCAB = 4096
N_TOK = 8 * 2048
NC, NS, L = 2, 16, 16
NW = NC * NS
TPW = N_TOK // NW
CH = 4
NBUF = 4
NCHUNK = TPW // CH
NOUTER = NCHUNK // NBUF


_LSE_RB = 256


def _lse_body(tbl_ref, lse_ref):
    x = tbl_ref[...]
    m = jnp.max(x, axis=-1, keepdims=True)
    s = jnp.sum(jnp.exp(x - m), axis=-1, keepdims=True)
    lse_ref[...] = m + jnp.log(s)


_lse_call = pl.pallas_call(
    _lse_body,
    grid=(VOCAB // _LSE_RB,),
    in_specs=[pl.BlockSpec((_LSE_RB, VOCAB), lambda i: (i, 0))],
    out_specs=pl.BlockSpec((_LSE_RB, 1), lambda i: (i, 0)),
    out_shape=jax.ShapeDtypeStruct((VOCAB, 1), jnp.float32),
)


_sc_mesh = plsc.VectorSubcoreMesh(
    core_axis_name="c", subcore_axis_name="s", num_cores=NC, num_subcores=NS
)


@functools.partial(
    pl.kernel,
    out_type=(
        jax.ShapeDtypeStruct((N_TOK, VOCAB), jnp.float32),
        jax.ShapeDtypeStruct((NW, L), jnp.float32),
    ),
    mesh=_sc_mesh,
    compiler_params=pltpu.CompilerParams(needs_layout_passes=False),
    scratch_types=[
        pltpu.VMEM((NBUF, CH, VOCAB), jnp.float32),
        pltpu.VMEM((NCHUNK, CH), jnp.int32),
        pltpu.VMEM((NCHUNK, CH), jnp.int32),
        pltpu.VMEM((VOCAB,), jnp.float32),
        pltpu.VMEM((L,), jnp.float32),
        pltpu.SemaphoreType.DMA,
        pltpu.SemaphoreType.DMA,
        pltpu.SemaphoreType.DMA,
        pltpu.SemaphoreType.DMA,
        pltpu.SemaphoreType.DMA,
        pltpu.SemaphoreType.DMA,
        pltpu.SemaphoreType.DMA,
        pltpu.SemaphoreType.DMA,
    ],
)
def _sc_call(table, idxg, tgt, lse, logits_out, part_out,
             rows_v, idxg_v, tgt_v, lse_v, acc_v,
             g0, g1, g2, g3, w0, w1, w2, w3):
    gsems = (g0, g1, g2, g3)
    wsems = (w0, w1, w2, w3)
    wid = lax.axis_index("s") * NC + lax.axis_index("c")
    base = wid * TPW

    def g_start(c, slot):
        pltpu.async_copy(table.at[idxg_v.at[c]], rows_v.at[slot], gsems[slot])

    def g_wait(c, slot):
        pltpu.make_async_copy(
            table.at[idxg_v.at[c]], rows_v.at[slot], gsems[slot]
        ).wait()

    def w_start(c, slot):
        pltpu.async_copy(
            rows_v.at[slot],
            logits_out.at[pl.ds(base + c * CH, CH)],
            wsems[slot],
        )

    def w_wait(c, slot):
        pltpu.make_async_copy(
            rows_v.at[slot],
            logits_out.at[pl.ds(base + c * CH, CH)],
            wsems[slot],
        ).wait()

    pltpu.sync_copy(idxg.at[wid], idxg_v)
    g_start(0, 0)
    g_start(1, 1)
    pltpu.sync_copy(tgt.at[wid], tgt_v)
    pltpu.sync_copy(lse, lse_v)

    lanes = lax.iota(jnp.int32, L)
    rvec = lanes % CH
    lane_ok = lanes < CH

    def outer(t, acc):
        for b in range(NBUF):
            g = t * NBUF + b
            g_wait(g, b)
            w_start(g, b)
            cvec = jnp.full((L,), g, jnp.int32)
            ivec = plsc.load_gather(idxg_v, [cvec, rvec])
            tvec = plsc.load_gather(tgt_v, [cvec, rvec])
            lvec = plsc.load_gather(lse_v, [ivec])
            svec = jnp.full((L,), b, jnp.int32)
            pvec = plsc.load_gather(rows_v, [svec, rvec, tvec])
            acc = acc + jnp.where(lane_ok, lvec - pvec, 0.0)
            nb = (b + 2) % NBUF
            if b < 2:
                @pl.when(t >= 1)
                def _():
                    w_wait(g - 2, nb)
                g_start(g + 2, nb)
            else:
                @pl.when(t < NOUTER - 1)
                def _():
                    w_wait(g - 2, nb)
                    g_start(g + 2, nb)
        return acc

    acc = lax.fori_loop(0, NOUTER, outer, jnp.zeros((L,), jnp.float32))
    for b in range(NBUF):
        w_wait(NCHUNK - NBUF + b, b)

    acc_v[...] = acc
    pltpu.sync_copy(acc_v, part_out.at[wid])


def kernel(idx, targets, table):
    idx_flat = idx.reshape(-1).astype(jnp.int32)
    tgt_flat = targets.reshape(-1).astype(jnp.int32)
    lse = _lse_call(table).reshape(VOCAB)
    idxg = idx_flat.reshape(NW, NCHUNK, CH)
    tgt3 = tgt_flat.reshape(NW, NCHUNK, CH)
    logits, partials = _sc_call(table, idxg, tgt3, lse)
    loss = jnp.sum(partials) * (1.0 / N_TOK)
    return (logits, loss)


# speedup vs baseline: 2.4314x; 1.0282x over previous
import functools

import jax
import jax.numpy as jnp
from jax import lax
from jax.experimental import pallas as pl
from jax.experimental.pallas import tpu as pltpu
from jax.experimental.pallas import tpu_sc as plsc

VOCAB = 4096
N_TOK = 8 * 2048
NC, NS, L = 2, 16, 16
NW = NC * NS
TPW = N_TOK // NW
CH = 4
NBUF = 4
NCHUNK = TPW // CH
NOUTER = NCHUNK // NBUF


_LSE_RB = 256


def _lse_body(tbl_ref, lse_ref):
    x = tbl_ref[...]
    m = jnp.max(x, axis=-1, keepdims=True)
    s = jnp.sum(jnp.exp(x - m), axis=-1, keepdims=True)
    lse_ref[...] = m + jnp.log(s)


_lse_call = pl.pallas_call(
    _lse_body,
    grid=(VOCAB // _LSE_RB,),
    in_specs=[pl.BlockSpec((_LSE_RB, VOCAB), lambda i: (i, 0))],
    out_specs=pl.BlockSpec((_LSE_RB, 1), lambda i: (i, 0)),
    out_shape=jax.ShapeDtypeStruct((VOCAB, 1), jnp.float32),
)


_HV = 512
_HI = 512


def _lsesum_body(idx_ref, lse_ref, out_ref):
    step = pl.program_id(0)
    vrow = lax.broadcasted_iota(jnp.int32, (1, _HV), 1) + step * _HV
    h = jnp.zeros((1, _HV), jnp.float32)
    for j in range(N_TOK // _HI):
        idxs = idx_ref[pl.ds(j * _HI, _HI), :]
        eq = (idxs == vrow).astype(jnp.float32)
        h = h + jnp.sum(eq, axis=0, keepdims=True)
    out_ref[...] = jnp.sum(h * lse_ref[0]).reshape(1, 1, 1)


_lsesum_call = pl.pallas_call(
    _lsesum_body,
    grid=(VOCAB // _HV,),
    in_specs=[
        pl.BlockSpec((N_TOK, 1), lambda i: (0, 0)),
        pl.BlockSpec((1, 1, _HV), lambda i: (i, 0, 0)),
    ],
    out_specs=pl.BlockSpec((1, 1, 1), lambda i: (i, 0, 0)),
    out_shape=jax.ShapeDtypeStruct((VOCAB // _HV, 1, 1), jnp.float32),
)


_sc_mesh = plsc.VectorSubcoreMesh(
    core_axis_name="c", subcore_axis_name="s", num_cores=NC, num_subcores=NS
)


@functools.partial(
    pl.kernel,
    out_type=(
        jax.ShapeDtypeStruct((N_TOK, VOCAB), jnp.float32),
        jax.ShapeDtypeStruct((NW, L), jnp.float32),
    ),
    mesh=_sc_mesh,
    compiler_params=pltpu.CompilerParams(needs_layout_passes=False),
    scratch_types=[
        pltpu.VMEM((NBUF, CH, VOCAB), jnp.float32),
        pltpu.VMEM((NCHUNK, CH), jnp.int32),
        pltpu.VMEM((NCHUNK, CH), jnp.int32),
        pltpu.VMEM((L,), jnp.float32),
        pltpu.SemaphoreType.DMA,
        pltpu.SemaphoreType.DMA,
        pltpu.SemaphoreType.DMA,
        pltpu.SemaphoreType.DMA,
        pltpu.SemaphoreType.DMA,
        pltpu.SemaphoreType.DMA,
        pltpu.SemaphoreType.DMA,
        pltpu.SemaphoreType.DMA,
    ],
)
def _sc_call(table, idxg, tgt, logits_out, part_out,
             rows_v, idxg_v, tgt_v, acc_v,
             g0, g1, g2, g3, w0, w1, w2, w3):
    gsems = (g0, g1, g2, g3)
    wsems = (w0, w1, w2, w3)
    wid = lax.axis_index("s") * NC + lax.axis_index("c")
    base = wid * TPW

    def g_start(c, slot):
        pltpu.async_copy(table.at[idxg_v.at[c]], rows_v.at[slot], gsems[slot])

    def g_wait(c, slot):
        pltpu.make_async_copy(
            table.at[idxg_v.at[c]], rows_v.at[slot], gsems[slot]
        ).wait()

    def w_start(c, slot):
        pltpu.async_copy(
            rows_v.at[slot],
            logits_out.at[pl.ds(base + c * CH, CH)],
            wsems[slot],
        )

    def w_wait(c, slot):
        pltpu.make_async_copy(
            rows_v.at[slot],
            logits_out.at[pl.ds(base + c * CH, CH)],
            wsems[slot],
        ).wait()

    pltpu.sync_copy(idxg.at[wid], idxg_v)
    g_start(0, 0)
    g_start(1, 1)
    pltpu.sync_copy(tgt.at[wid], tgt_v)

    lanes = lax.iota(jnp.int32, L)
    rvec = lanes % CH
    lane_ok = lanes < CH

    def outer(t, acc):
        for b in range(NBUF):
            g = t * NBUF + b
            g_wait(g, b)
            w_start(g, b)
            cvec = jnp.full((L,), g, jnp.int32)
            tvec = plsc.load_gather(tgt_v, [cvec, rvec])
            svec = jnp.full((L,), b, jnp.int32)
            pvec = plsc.load_gather(rows_v, [svec, rvec, tvec])
            acc = acc + jnp.where(lane_ok, pvec, 0.0)
            nb = (b + 2) % NBUF
            if b < 2:
                @pl.when(t >= 1)
                def _():
                    w_wait(g - 2, nb)
                g_start(g + 2, nb)
            else:
                @pl.when(t < NOUTER - 1)
                def _():
                    w_wait(g - 2, nb)
                    g_start(g + 2, nb)
        return acc

    acc = lax.fori_loop(0, NOUTER, outer, jnp.zeros((L,), jnp.float32))
    for b in range(NBUF):
        w_wait(NCHUNK - NBUF + b, b)

    acc_v[...] = acc
    pltpu.sync_copy(acc_v, part_out.at[wid])


def kernel(idx, targets, table):
    idx_flat = idx.reshape(-1).astype(jnp.int32)
    tgt_flat = targets.reshape(-1).astype(jnp.int32)
    idxg = idx_flat.reshape(NW, NCHUNK, CH)
    tgt3 = tgt_flat.reshape(NW, NCHUNK, CH)
    logits, picked_parts = _sc_call(table, idxg, tgt3)
    lse = _lse_call(table)
    lse_parts = _lsesum_call(idx_flat.reshape(N_TOK, 1), lse.reshape(
        VOCAB // _HV, 1, _HV))
    loss = (jnp.sum(lse_parts) - jnp.sum(picked_parts)) * (1.0 / N_TOK)
    return (logits, loss)


# speedup vs baseline: 2.5519x; 1.0496x over previous
import functools

import jax
import jax.numpy as jnp
from jax import lax
from jax.experimental import pallas as pl
from jax.experimental.pallas import tpu as pltpu
from jax.experimental.pallas import tpu_sc as plsc

VOCAB = 4096
N_TOK = 8 * 2048
NC, NS, L = 2, 16, 16
NW = NC * NS
TPW = N_TOK // NW
CH = 8
NBUF = 3
NCHUNK = TPW // CH
NOUTER = (NCHUNK - 1) // NBUF


_LSE_RB = 256


def _lse_body(tbl_ref, lse_ref):
    x = tbl_ref[...]
    m = jnp.max(x, axis=-1, keepdims=True)
    s = jnp.sum(jnp.exp(x - m), axis=-1, keepdims=True)
    lse_ref[...] = m + jnp.log(s)


_lse_call = pl.pallas_call(
    _lse_body,
    grid=(VOCAB // _LSE_RB,),
    in_specs=[pl.BlockSpec((_LSE_RB, VOCAB), lambda i: (i, 0))],
    out_specs=pl.BlockSpec((_LSE_RB, 1), lambda i: (i, 0)),
    out_shape=jax.ShapeDtypeStruct((VOCAB, 1), jnp.float32),
)


_HV = 512
_HR = 128


def _lsesum_body(idx_ref, lse_ref, out_ref):
    step = pl.program_id(0)
    vcol = lax.broadcasted_iota(jnp.int32, (_HV, 1), 0) + step * _HV

    def body(r, h):
        row = idx_ref[pl.ds(r, 1), :]
        return h + (row == vcol).astype(jnp.float32)

    h = lax.fori_loop(0, _HR, body, jnp.zeros((_HV, 128), jnp.float32))
    hsum = jnp.sum(h, axis=1, keepdims=True)
    out_ref[...] = jnp.sum(hsum * lse_ref[...]).reshape(1, 1, 1)


_lsesum_call = pl.pallas_call(
    _lsesum_body,
    grid=(VOCAB // _HV,),
    in_specs=[
        pl.BlockSpec((_HR, 128), lambda i: (0, 0)),
        pl.BlockSpec((_HV, 1), lambda i: (i, 0)),
    ],
    out_specs=pl.BlockSpec((1, 1, 1), lambda i: (i, 0, 0)),
    out_shape=jax.ShapeDtypeStruct((VOCAB // _HV, 1, 1), jnp.float32),
)


_sc_mesh = plsc.VectorSubcoreMesh(
    core_axis_name="c", subcore_axis_name="s", num_cores=NC, num_subcores=NS
)


@functools.partial(
    pl.kernel,
    out_type=(
        jax.ShapeDtypeStruct((N_TOK, VOCAB), jnp.float32),
        jax.ShapeDtypeStruct((NW, L), jnp.float32),
    ),
    mesh=_sc_mesh,
    compiler_params=pltpu.CompilerParams(needs_layout_passes=False),
    scratch_types=[
        pltpu.VMEM((NBUF, CH, VOCAB), jnp.float32),
        pltpu.VMEM((1, TPW), jnp.int32),
        pltpu.VMEM((1, TPW), jnp.int32),
        pltpu.VMEM((L,), jnp.float32),
        pltpu.SemaphoreType.DMA,
        pltpu.SemaphoreType.DMA,
        pltpu.SemaphoreType.DMA,
        pltpu.SemaphoreType.DMA,
        pltpu.SemaphoreType.DMA,
        pltpu.SemaphoreType.DMA,
    ],
)
def _sc_call(table, idx2, tgt2, logits_out, part_out,
             rows_v, idxg_v, tgt_v, acc_v,
             g0, g1, g2, w0, w1, w2):
    gsems = (g0, g1, g2)
    wsems = (w0, w1, w2)
    wid = lax.axis_index("s") * NC + lax.axis_index("c")
    base = wid * TPW

    def g_start(c, slot):
        pltpu.async_copy(
            table.at[idxg_v.at[0, pl.ds(c * CH, CH)]],
            rows_v.at[slot],
            gsems[slot],
        )

    def g_wait(c, slot):
        pltpu.make_async_copy(
            table.at[idxg_v.at[0, pl.ds(c * CH, CH)]],
            rows_v.at[slot],
            gsems[slot],
        ).wait()

    def w_start(c, slot):
        pltpu.async_copy(
            rows_v.at[slot],
            logits_out.at[pl.ds(base + c * CH, CH)],
            wsems[slot],
        )

    def w_wait(c, slot):
        pltpu.make_async_copy(
            rows_v.at[slot],
            logits_out.at[pl.ds(base + c * CH, CH)],
            wsems[slot],
        ).wait()

    pltpu.sync_copy(idx2.at[pl.ds(wid, 1)], idxg_v)
    g_start(0, 0)
    g_start(1, 1)
    pltpu.sync_copy(tgt2.at[pl.ds(wid, 1)], tgt_v)

    lanes = lax.iota(jnp.int32, L)
    rvec = lanes % CH
    lane_ok = lanes < CH
    zeros = jnp.zeros((L,), jnp.int32)

    def pick(g, slot, acc):
        cols = g * CH + rvec
        tvec = plsc.load_gather(tgt_v, [zeros, cols])
        svec = jnp.full((L,), slot, jnp.int32)
        pvec = plsc.load_gather(rows_v, [svec, rvec, tvec])
        return acc + jnp.where(lane_ok, pvec, 0.0)

    def outer(t, acc):
        for b in range(NBUF):
            g = t * NBUF + b
            g_wait(g, b)
            w_start(g, b)
            acc = pick(g, b, acc)
            ns = (b + 2) % NBUF
            if b == 0:
                @pl.when(t >= 1)
                def _():
                    w_wait(g - 1, ns)
                g_start(g + 2, ns)
            elif b == 1:
                w_wait(g - 1, ns)
                g_start(g + 2, ns)
            else:
                @pl.when(t < NOUTER - 1)
                def _():
                    w_wait(g - 1, ns)
                    g_start(g + 2, ns)
        return acc

    acc = lax.fori_loop(0, NOUTER, outer, jnp.zeros((L,), jnp.float32))
    gl = NCHUNK - 1
    g_wait(gl, 0)
    w_start(gl, 0)
    acc = pick(gl, 0, acc)
    w_wait(gl - 2, 1)
    w_wait(gl - 1, 2)
    w_wait(gl, 0)

    acc_v[...] = acc
    pltpu.sync_copy(acc_v, part_out.at[wid])


def kernel(idx, targets, table):
    idx_flat = idx.reshape(-1).astype(jnp.int32)
    tgt_flat = targets.reshape(-1).astype(jnp.int32)
    idx2 = idx_flat.reshape(NW, TPW)
    tgt2 = tgt_flat.reshape(NW, TPW)
    logits, picked_parts = _sc_call(table, idx2, tgt2)
    lse = _lse_call(table)
    lse_parts = _lsesum_call(idx_flat.reshape(_HR, 128), lse)
    loss = (jnp.sum(lse_parts) - jnp.sum(picked_parts)) * (1.0 / N_TOK)
    return (logits, loss)


# speedup vs baseline: 2.5906x; 1.0152x over previous
import functools

import jax
import jax.numpy as jnp
from jax import lax
from jax.experimental import pallas as pl
from jax.experimental.pallas import tpu as pltpu
from jax.experimental.pallas import tpu_sc as plsc

VOCAB = 4096
N_TOK = 8 * 2048
NC, NS, L = 2, 16, 16
NW = NC * NS
TPW = N_TOK // NW
CH = 8
NBUF = 3
NCHUNK = TPW // CH
NOUTER = (NCHUNK - 1) // NBUF


_LSE_RB = 256


def _lse_body(tbl_ref, lse_ref):
    x = tbl_ref[...]
    m = jnp.max(x, axis=-1, keepdims=True)
    s = jnp.sum(jnp.exp(x - m), axis=-1, keepdims=True)
    lse_ref[...] = m + jnp.log(s)


_lse_call = pl.pallas_call(
    _lse_body,
    grid=(VOCAB // _LSE_RB,),
    in_specs=[pl.BlockSpec((_LSE_RB, VOCAB), lambda i: (i, 0))],
    out_specs=pl.BlockSpec((_LSE_RB, 1), lambda i: (i, 0)),
    out_shape=jax.ShapeDtypeStruct((VOCAB, 1), jnp.float32),
)


_HV = 512
_HR = 128


def _lsesum_body(idx_ref, lse_ref, out_ref):
    step = pl.program_id(0)
    vcol = lax.broadcasted_iota(jnp.int32, (_HV, 1), 0) + step * _HV

    def body(r, h):
        row = idx_ref[pl.ds(r, 1), :]
        return h + (row == vcol).astype(jnp.float32)

    h = lax.fori_loop(0, _HR, body, jnp.zeros((_HV, 128), jnp.float32))
    hsum = jnp.sum(h, axis=1, keepdims=True)
    out_ref[...] = jnp.sum(hsum * lse_ref[...]).reshape(1, 1, 1)


_lsesum_call = pl.pallas_call(
    _lsesum_body,
    grid=(VOCAB // _HV,),
    in_specs=[
        pl.BlockSpec((_HR, 128), lambda i: (0, 0)),
        pl.BlockSpec((_HV, 1), lambda i: (i, 0)),
    ],
    out_specs=pl.BlockSpec((1, 1, 1), lambda i: (i, 0, 0)),
    out_shape=jax.ShapeDtypeStruct((VOCAB // _HV, 1, 1), jnp.float32),
)


_sc_mesh = plsc.VectorSubcoreMesh(
    core_axis_name="c", subcore_axis_name="s", num_cores=NC, num_subcores=NS
)


@functools.partial(
    pl.kernel,
    out_type=(
        jax.ShapeDtypeStruct((N_TOK, VOCAB), jnp.float32),
        jax.ShapeDtypeStruct((NW, L), jnp.float32),
    ),
    mesh=_sc_mesh,
    compiler_params=pltpu.CompilerParams(needs_layout_passes=False),
    scratch_types=[
        pltpu.VMEM((NBUF, CH, VOCAB), jnp.float32),
        pltpu.VMEM((1, TPW), jnp.int32),
        pltpu.VMEM((1, TPW), jnp.int32),
        pltpu.VMEM((L,), jnp.float32),
        pltpu.SemaphoreType.DMA,
        pltpu.SemaphoreType.DMA,
        pltpu.SemaphoreType.DMA,
        pltpu.SemaphoreType.DMA,
        pltpu.SemaphoreType.DMA,
        pltpu.SemaphoreType.DMA,
    ],
)
def _sc_call(table, idx2, tgt2, logits_out, part_out,
             rows_v, idxg_v, tgt_v, acc_v,
             g0, g1, g2, w0, w1, w2):
    gsems = (g0, g1, g2)
    wsems = (w0, w1, w2)
    wid = lax.axis_index("s") * NC + lax.axis_index("c")
    base = wid * TPW

    def g_start(c, slot):
        pltpu.async_copy(
            table.at[idxg_v.at[0, pl.ds(c * CH, CH)]],
            rows_v.at[slot],
            gsems[slot],
        )

    def g_wait(c, slot):
        pltpu.make_async_copy(
            table.at[idxg_v.at[0, pl.ds(c * CH, CH)]],
            rows_v.at[slot],
            gsems[slot],
        ).wait()

    def w_start(c, slot):
        pltpu.async_copy(
            rows_v.at[slot],
            logits_out.at[pl.ds(base + c * CH, CH)],
            wsems[slot],
        )

    def w_wait(c, slot):
        pltpu.make_async_copy(
            rows_v.at[slot],
            logits_out.at[pl.ds(base + c * CH, CH)],
            wsems[slot],
        ).wait()

    pltpu.sync_copy(idx2.at[pl.ds(wid, 1)], idxg_v)
    g_start(0, 0)
    g_start(1, 1)
    pltpu.sync_copy(tgt2.at[pl.ds(wid, 1)], tgt_v)

    lanes = lax.iota(jnp.int32, L)
    rvec = lanes % CH
    lane_ok = lanes < CH
    zeros = jnp.zeros((L,), jnp.int32)

    def pick(g, slot, acc):
        cols = g * CH + rvec
        tvec = plsc.load_gather(tgt_v, [zeros, cols])
        svec = jnp.full((L,), slot, jnp.int32)
        pvec = plsc.load_gather(rows_v, [svec, rvec, tvec])
        return acc + jnp.where(lane_ok, pvec, 0.0)

    def outer(t, acc):
        for b in range(NBUF):
            g = t * NBUF + b
            g_wait(g, b)
            w_start(g, b)
            ns = (b + 2) % NBUF
            if b == 0:
                @pl.when(t >= 1)
                def _():
                    w_wait(g - 1, ns)
                g_start(g + 2, ns)
            elif b == 1:
                w_wait(g - 1, ns)
                g_start(g + 2, ns)
            else:
                @pl.when(t < NOUTER - 1)
                def _():
                    w_wait(g - 1, ns)
                    g_start(g + 2, ns)
            acc = pick(g, b, acc)
        return acc

    acc = lax.fori_loop(0, NOUTER, outer, jnp.zeros((L,), jnp.float32))
    gl = NCHUNK - 1
    g_wait(gl, 0)
    w_start(gl, 0)
    acc = pick(gl, 0, acc)
    w_wait(gl - 2, 1)
    w_wait(gl - 1, 2)
    w_wait(gl, 0)

    acc_v[...] = acc
    pltpu.sync_copy(acc_v, part_out.at[wid])


def kernel(idx, targets, table):
    idx_flat = idx.reshape(-1).astype(jnp.int32)
    tgt_flat = targets.reshape(-1).astype(jnp.int32)
    idx2 = idx_flat.reshape(NW, TPW)
    tgt2 = tgt_flat.reshape(NW, TPW)
    logits, picked_parts = _sc_call(table, idx2, tgt2)
    lse = _lse_call(table)
    lse_parts = _lsesum_call(idx_flat.reshape(_HR, 128), lse)
    loss = (jnp.sum(lse_parts) - jnp.sum(picked_parts)) * (1.0 / N_TOK)
    return (logits, loss)
